# Initial kernel scaffold; baseline (speedup 1.0000x reference)
import jax
import jax.numpy as jnp
from jax.experimental import pallas as pl


def kernel(x, noise):
    raise NotImplementedError("write your pallas kernel here")


# speedup vs baseline: 2.0409x; 2.0409x over previous
import jax
import jax.numpy as jnp
from jax.experimental import pallas as pl

_WIN = 16
_NW = 6
_NWIN = _NW * _NW * _NW
_LEN_KEEP = int(_NWIN * (1 - 0.75))


def _mask_kernel(nrow_ref, ncol_ref, x_ref, xm_ref, mask_ref):
    i = pl.program_id(1)
    j = pl.program_id(2)
    dc = x_ref.shape[-1]
    seg = dc // _NW

    nj = nrow_ref[0]
    ni = ncol_ref[0]
    nj_m = jnp.broadcast_to(nj, (_NWIN, _NWIN))
    ni_m = jnp.broadcast_to(ni, (_NWIN, _NWIN))
    row = jax.lax.broadcasted_iota(jnp.int32, (_NWIN, _NWIN), 0)
    col = jax.lax.broadcasted_iota(jnp.int32, (_NWIN, _NWIN), 1)
    less = (nj_m < ni_m) | ((nj_m == ni_m) & (col < row))
    rank = jnp.sum(less.astype(jnp.float32), axis=1, keepdims=True)
    keep = (rank < _LEN_KEEP).astype(jnp.float32)

    base = i * (_NW * _NW) + j * _NW
    lane = jax.lax.broadcasted_iota(jnp.int32, (_NWIN, dc), 1)
    wrow = jax.lax.broadcasted_iota(jnp.int32, (_NWIN, dc), 0)
    onehot = (wrow == base + lane // seg).astype(jnp.float32)
    keep_lane = jnp.sum(keep * onehot, axis=0, keepdims=True)
    keep_b = keep_lane.reshape(1, 1, 1, dc)
    mask_ref[...] = jnp.broadcast_to(1.0 - keep_b, mask_ref.shape)
    xm_ref[...] = x_ref[...] * keep_b


def kernel(x, noise):
    B, H, W, D, C = x.shape
    dc = D * C
    xr = x.reshape(B, H, W, dc)
    nrow = noise.reshape(B, 1, _NWIN)
    ncol = noise.reshape(B, _NWIN, 1)

    grid = (B, H // _WIN, W // _WIN)
    blk = (1, _WIN, _WIN, dc)
    xm, mask = pl.pallas_call(
        _mask_kernel,
        grid=grid,
        in_specs=[
            pl.BlockSpec((1, 1, _NWIN), lambda b, i, j: (b, 0, 0)),
            pl.BlockSpec((1, _NWIN, 1), lambda b, i, j: (b, 0, 0)),
            pl.BlockSpec(blk, lambda b, i, j: (b, i, j, 0)),
        ],
        out_specs=[
            pl.BlockSpec(blk, lambda b, i, j: (b, i, j, 0)),
            pl.BlockSpec(blk, lambda b, i, j: (b, i, j, 0)),
        ],
        out_shape=[
            jax.ShapeDtypeStruct((B, H, W, dc), x.dtype),
            jax.ShapeDtypeStruct((B, H, W, dc), x.dtype),
        ],
    )(nrow, ncol, xr)
    return xm.reshape(B, H, W, D, C), mask.reshape(B, H, W, D, C)
